# baseline (speedup 1.0000x reference)
import functools

import jax
import jax.numpy as jnp
from jax import lax
from jax.experimental import pallas as pl
from jax.experimental.pallas import tpu as pltpu
from jax.experimental.pallas import tpu_sc as plsc

_LANES = 16


def _build_lookup(B, V, D, num_workers, chunk, nbuf):
    b_per_w = B // num_workers
    n_chunks = b_per_w // chunk
    n_outer = n_chunks // nbuf
    assert n_outer * nbuf == n_chunks
    scale = float(D) ** 0.5
    mesh = plsc.VectorSubcoreMesh(core_axis_name="c", subcore_axis_name="s")
    nc = 2
    D2 = 2 * D

    scratch = [pltpu.VMEM((b_per_w,), jnp.int32)]
    scratch += [pltpu.VMEM((chunk, D), jnp.float32) for _ in range(nbuf)]
    scratch += [pltpu.VMEM((chunk, D2), jnp.float32) for _ in range(nbuf)]
    scratch += [pltpu.SemaphoreType.DMA for _ in range(2 * nbuf)]

    @functools.partial(
        pl.kernel,
        mesh=mesh,
        out_type=jax.ShapeDtypeStruct((B, D2), jnp.float32),
        scratch_types=scratch,
        compiler_params=pltpu.CompilerParams(
            use_tc_tiling_on_sc=False, needs_layout_passes=False
        ),
    )
    def lookup(idx_hbm, table_hbm, out_hbm, idx_v, *bufs):
        gb = bufs[:nbuf]
        sb = bufs[nbuf : 2 * nbuf]
        sem_g = bufs[2 * nbuf : 3 * nbuf]
        sem_s = bufs[3 * nbuf :]

        wid = lax.axis_index("s") * nc + lax.axis_index("c")
        base = wid * b_per_w
        pltpu.sync_copy(idx_hbm.at[pl.ds(base, b_per_w)], idx_v)

        def dbl(k, c2):
            s = pl.ds(k * _LANES, _LANES)
            v = idx_v[s]
            idx_v[s] = v + v
            return c2

        lax.fori_loop(0, b_per_w // _LANES, dbl, 0, unroll=8)

        def gather(off, b):
            return pltpu.make_async_copy(
                table_hbm.at[idx_v.at[pl.ds(off, chunk)]], gb[b], sem_g[b]
            )

        def put(b, off):
            return pltpu.make_async_copy(
                sb[b], out_hbm.at[pl.ds(base + off, chunk)], sem_s[b]
            )

        for b in range(nbuf):
            gather(b * chunk, b).start()

        def outer_body(o, carry):
            for b in range(nbuf):
                off = (o * nbuf + b) * chunk
                gather(off, b).wait()

                @pl.when(o > 0)
                def _wait_prev_store():
                    put(b, 0).wait()

                def row_body(r, c2):
                    for c in range(D // _LANES):
                        s = pl.ds(c * _LANES, _LANES)
                        sb[b][r, s] = gb[b][r, s] * scale
                    return c2

                lax.fori_loop(0, chunk, row_body, 0, unroll=4)
                put(b, off).start()

                @pl.when(off + nbuf * chunk < b_per_w)
                def _next_gather():
                    gather(off + nbuf * chunk, b).start()

            return carry

        lax.fori_loop(0, n_outer, outer_body, 0)

        for b in range(nbuf):
            put(b, 0).wait()

    return lookup


def kernel(input_, table):
    B0, S = input_.shape
    V, D = table.shape
    B = B0 * S
    idx = input_.reshape(B).astype(jnp.int32)
    table_d = jnp.broadcast_to(table[:, None, :], (V, 2, D)).reshape(2 * V, D)
    lookup = _build_lookup(B, V, D, num_workers=32, chunk=64, nbuf=4)
    out_p = lookup(idx, table_d)
    return out_p[:, :D].reshape(B0, S, D)


# speedup vs baseline: 1.7984x; 1.7984x over previous
import functools

import jax
import jax.numpy as jnp
from jax import lax
from jax.experimental import pallas as pl
from jax.experimental.pallas import tpu as pltpu
from jax.experimental.pallas import tpu_sc as plsc

_LANES = 16


def _build_lookup(B, V, D, num_workers, chunk, nbuf):
    b_per_w = B // num_workers
    n_chunks = b_per_w // chunk
    n_outer = n_chunks // nbuf
    assert n_outer * nbuf == n_chunks
    scale = float(D) ** 0.5
    mesh = plsc.VectorSubcoreMesh(core_axis_name="c", subcore_axis_name="s")
    nc = 2
    D2 = 2 * D

    scratch = [pltpu.VMEM((b_per_w,), jnp.int32)]
    scratch += [pltpu.VMEM((chunk, D), jnp.float32) for _ in range(nbuf)]
    scratch += [pltpu.VMEM((chunk, D2), jnp.float32) for _ in range(nbuf)]
    scratch += [pltpu.SemaphoreType.DMA for _ in range(2 * nbuf)]

    @functools.partial(
        pl.kernel,
        mesh=mesh,
        out_type=jax.ShapeDtypeStruct((B, D2), jnp.float32),
        scratch_types=scratch,
        compiler_params=pltpu.CompilerParams(
            use_tc_tiling_on_sc=False, needs_layout_passes=False
        ),
    )
    def lookup(idx_hbm, table_hbm, out_hbm, idx_v, *bufs):
        gb = bufs[:nbuf]
        sb = bufs[nbuf : 2 * nbuf]
        sem_g = bufs[2 * nbuf : 3 * nbuf]
        sem_s = bufs[3 * nbuf :]

        wid = lax.axis_index("s") * nc + lax.axis_index("c")
        base = wid * b_per_w
        pltpu.sync_copy(idx_hbm.at[pl.ds(base, b_per_w)], idx_v)

        def dbl(k, c2):
            s = pl.ds(k * _LANES, _LANES)
            v = idx_v[s]
            idx_v[s] = v + v
            return c2

        lax.fori_loop(0, b_per_w // _LANES, dbl, 0, unroll=8)

        def gather(off, b):
            return pltpu.make_async_copy(
                table_hbm.at[idx_v.at[pl.ds(off, chunk)]], gb[b], sem_g[b]
            )

        def put(b, off):
            return pltpu.make_async_copy(
                sb[b], out_hbm.at[pl.ds(base + off, chunk)], sem_s[b]
            )

        for b in range(nbuf):
            gather(b * chunk, b).start()

        def outer_body(o, carry):
            for b in range(nbuf):
                off = (o * nbuf + b) * chunk
                gather(off, b).wait()

                @pl.when(o > 0)
                def _wait_prev_store():
                    put(b, 0).wait()

                def row_body(r, c2):
                    for c in range(D // _LANES):
                        s = pl.ds(c * _LANES, _LANES)
                        sb[b][r, s] = gb[b][r, s] * scale
                    return c2

                lax.fori_loop(0, chunk, row_body, 0, unroll=4)
                put(b, off).start()

                @pl.when(off + nbuf * chunk < b_per_w)
                def _next_gather():
                    gather(off + nbuf * chunk, b).start()

            return carry

        lax.fori_loop(0, n_outer, outer_body, 0)

        for b in range(nbuf):
            put(b, 0).wait()

    return lookup


def kernel(input_, table):
    B0, S = input_.shape
    V, D = table.shape
    B = B0 * S
    idx = input_.reshape(B).astype(jnp.int32)
    table_d = jnp.pad(table, ((0, 0), (0, D))).reshape(2 * V, D)
    lookup = _build_lookup(B, V, D, num_workers=32, chunk=64, nbuf=4)
    out_p = lookup(idx, table_d)
    return out_p[:, :D].reshape(B0, S, D)


# speedup vs baseline: 2.2541x; 1.2534x over previous
import functools

import jax
import jax.numpy as jnp
from jax import lax
from jax.experimental import pallas as pl
from jax.experimental.pallas import tpu as pltpu
from jax.experimental.pallas import tpu_sc as plsc

_LANES = 16


def _build_lookup(B, V, D, num_workers, chunk, nbuf):
    b_per_w = B // num_workers
    n_chunks = b_per_w // chunk
    n_outer = n_chunks // nbuf
    assert n_outer * nbuf == n_chunks
    scale = float(D) ** 0.5
    mesh = plsc.VectorSubcoreMesh(core_axis_name="c", subcore_axis_name="s")
    nc = 2
    D2 = 2 * D

    scratch = [pltpu.VMEM((b_per_w,), jnp.int32)]
    scratch += [pltpu.VMEM((chunk, D), jnp.float32) for _ in range(nbuf)]
    scratch += [pltpu.VMEM((chunk, D2), jnp.float32) for _ in range(nbuf)]
    scratch += [pltpu.SemaphoreType.DMA for _ in range(2 * nbuf)]

    @functools.partial(
        pl.kernel,
        mesh=mesh,
        out_type=jax.ShapeDtypeStruct((B, D2), jnp.float32),
        scratch_types=scratch,
        compiler_params=pltpu.CompilerParams(
            use_tc_tiling_on_sc=False, needs_layout_passes=False
        ),
    )
    def lookup(idx_hbm, table_hbm, out_hbm, idx_v, *bufs):
        gb = bufs[:nbuf]
        sb = bufs[nbuf : 2 * nbuf]
        sem_g = bufs[2 * nbuf : 3 * nbuf]
        sem_s = bufs[3 * nbuf :]

        wid = lax.axis_index("s") * nc + lax.axis_index("c")
        base = wid * b_per_w
        pltpu.sync_copy(idx_hbm.at[pl.ds(base, b_per_w)], idx_v)

        @plsc.parallel_loop(0, b_per_w // _LANES, unroll=8)
        def _dbl(k):
            s = pl.ds(k * _LANES, _LANES)
            v = idx_v[s]
            idx_v[s] = v + v

        def gather(off, b):
            return pltpu.make_async_copy(
                table_hbm.at[idx_v.at[pl.ds(off, chunk)]], gb[b], sem_g[b]
            )

        def put(b, off):
            return pltpu.make_async_copy(
                sb[b], out_hbm.at[pl.ds(base + off, chunk)], sem_s[b]
            )

        for b in range(nbuf):
            gather(b * chunk, b).start()

        def outer_body(o, carry):
            for b in range(nbuf):
                off = (o * nbuf + b) * chunk
                gather(off, b).wait()

                @pl.when(o > 0)
                def _wait_prev_store():
                    put(b, 0).wait()

                @plsc.parallel_loop(0, chunk, unroll=4)
                def _scale(r):
                    for c in range(D // _LANES):
                        s = pl.ds(c * _LANES, _LANES)
                        sb[b][r, s] = gb[b][r, s] * scale
                put(b, off).start()

                @pl.when(off + nbuf * chunk < b_per_w)
                def _next_gather():
                    gather(off + nbuf * chunk, b).start()

            return carry

        lax.fori_loop(0, n_outer, outer_body, 0)

        for b in range(nbuf):
            put(b, 0).wait()

    return lookup


def kernel(input_, table):
    B0, S = input_.shape
    V, D = table.shape
    B = B0 * S
    idx = input_.reshape(B).astype(jnp.int32)
    table_d = jnp.pad(table, ((0, 0), (0, D))).reshape(2 * V, D)
    lookup = _build_lookup(B, V, D, num_workers=32, chunk=64, nbuf=4)
    out_p = lookup(idx, table_d)
    return out_p[:, :D].reshape(B0, S, D)
